# baseline (speedup 1.0000x reference)
import functools

import jax
import jax.numpy as jnp
from jax import lax
from jax.experimental import pallas as pl
from jax.experimental.pallas import tpu as pltpu
from jax.experimental.pallas import tpu_sc as plsc

N = 131072
D = 64
P = 262144

K = 8
PK = P // K
NUM_WORKERS = 32
C = 256
HALF_C = C // 2
DIS_ROWS = 2 * PK
DW = DIS_ROWS // NUM_WORKERS
NCHD = DW // C
SA_ROWS = 2 * PK
SW = SA_ROWS // NUM_WORKERS
NCHS = SW // C

PREP_R = 16384
LOSS_B = 8192


def _prep_body(st_ref, nt_ref, combo_ref, part_ref):
    st = st_ref[...]
    dt = nt_ref[...] - st
    combo_ref[:, :D] = st.T
    combo_ref[:, D:] = dt.T
    part_ref[...] = jnp.full((1, 1, 128), jnp.sum(dt * dt), dtype=jnp.float32)


def _loss_body(gsi_ref, gsj_ref, w_ref, out_ref):
    i = pl.program_id(0)
    lane = lax.broadcasted_iota(jnp.int32, (LOSS_B, 2 * D), 1)
    first = lane < D

    gsi = gsi_ref[...]
    gsj = gsj_ref[...]
    q = gsi - gsj
    q2 = q * q
    s2 = jnp.sum(jnp.where(first, q2, 0.0), axis=1)
    dd2 = jnp.sum(q2, axis=1) - s2
    rep = jnp.sum(jnp.exp(-s2) * dd2)

    gsi2 = gsi * gsi
    gsj2 = gsj * gsj
    na2 = jnp.sum(jnp.where(first, 0.0, gsi2), axis=1)
    nb2 = jnp.sum(jnp.where(first, 0.0, gsj2), axis=1)
    prop = jnp.sum(na2 + nb2 - 2.0 * jnp.sqrt(na2 * nb2))

    l1 = jnp.where(i == 0, jnp.sum(jnp.abs(w_ref[...])), 0.0)

    out_lane = lax.broadcasted_iota(jnp.int32, (1, 1, 128), 2)
    out_ref[...] = jnp.where(
        out_lane == 1, prop,
        jnp.where(out_lane == 2, rep,
                  jnp.where(out_lane == 3, l1, 0.0)))


def _prep(states, next_states):
    grid = N // PREP_R
    return pl.pallas_call(
        _prep_body,
        grid=(grid,),
        in_specs=[
            pl.BlockSpec((D, PREP_R), lambda i: (0, i)),
            pl.BlockSpec((D, PREP_R), lambda i: (0, i)),
        ],
        out_specs=[
            pl.BlockSpec((PREP_R, 2 * D), lambda i: (i, 0)),
            pl.BlockSpec((1, 1, 128), lambda i: (i, 0, 0)),
        ],
        out_shape=[
            jax.ShapeDtypeStruct((N, 2 * D), jnp.float32),
            jax.ShapeDtypeStruct((grid, 1, 128), jnp.float32),
        ],
        compiler_params=pltpu.CompilerParams(
            dimension_semantics=("parallel",)),
    )(states.T, next_states.T)


def _loss(g_sa, W):
    grid = PK // LOSS_B
    nb = grid
    return pl.pallas_call(
        _loss_body,
        grid=(grid,),
        in_specs=[
            pl.BlockSpec((LOSS_B, 2 * D), lambda i: (i, 0)),
            pl.BlockSpec((LOSS_B, 2 * D), lambda i: (nb + i, 0)),
            pl.BlockSpec((D, D), lambda i: (0, 0)),
        ],
        out_specs=pl.BlockSpec((1, 1, 128), lambda i: (i, 0, 0)),
        out_shape=jax.ShapeDtypeStruct((grid, 1, 128), jnp.float32),
        compiler_params=pltpu.CompilerParams(
            dimension_semantics=("parallel",)),
    )(g_sa, g_sa, W)


def _dis_compute(rows_ref, acc_ref):
    @pl.loop(0, HALF_C // 16)
    def _(grp):
        lane16 = lax.iota(jnp.int32, 16)
        gs2 = jnp.zeros((16,), jnp.float32)
        for b in range(16):
            p = grp * 16 + b
            acc = None
            for c in range(D // 16):
                vi = rows_ref[p, pl.ds(c * 16, 16)]
                vj = rows_ref[p + HALF_C, pl.ds(c * 16, 16)]
                dlt = vi - vj
                acc = dlt * dlt if acc is None else acc + dlt * dlt
            s = jnp.sum(acc)
            gs2 = jnp.where(lane16 == b, s, gs2)
        acc_ref[...] = acc_ref[...] + jnp.exp(-gs2)


def _sc_pass(combo, idx_all):
    mesh = plsc.VectorSubcoreMesh(core_axis_name="c", subcore_axis_name="s")

    @functools.partial(
        pl.kernel,
        out_type=[
            jax.ShapeDtypeStruct((SA_ROWS, 2 * D), jnp.float32),
            jax.ShapeDtypeStruct((NUM_WORKERS, 16), jnp.float32),
        ],
        mesh=mesh,
        scratch_types=[
            pltpu.VMEM((C,), jnp.int32),
            pltpu.VMEM((C,), jnp.int32),
            pltpu.VMEM((C, 2 * D), jnp.float32),
            pltpu.VMEM((C, 2 * D), jnp.float32),
            pltpu.VMEM((16,), jnp.float32),
            pltpu.SemaphoreType.DMA,
            pltpu.SemaphoreType.DMA,
        ],
        compiler_params=pltpu.CompilerParams(needs_layout_passes=False),
    )
    def k(combo_hbm, idx_hbm, gsa_hbm, caus_hbm,
          idx_v0, idx_v1, rows_v0, rows_v1, acc_v, sem0, sem1):
        wid = lax.axis_index("s") * 2 + lax.axis_index("c")

        dbase = wid * DW
        acc_v[...] = jnp.zeros((16,), jnp.float32)
        pltpu.sync_copy(idx_hbm.at[pl.ds(dbase, C)], idx_v0)
        pltpu.async_copy(combo_hbm.at[idx_v0], rows_v0, sem0)

        @pl.loop(0, NCHD, step=2)
        def _(g):
            off1 = dbase + (g + 1) * C
            pltpu.sync_copy(idx_hbm.at[pl.ds(off1, C)], idx_v1)
            pltpu.async_copy(combo_hbm.at[idx_v1], rows_v1, sem1)
            pltpu.make_async_copy(combo_hbm.at[idx_v0], rows_v0, sem0).wait()
            _dis_compute(rows_v0, acc_v)

            @pl.when(g + 2 < NCHD)
            def _():
                off2 = dbase + (g + 2) * C
                pltpu.sync_copy(idx_hbm.at[pl.ds(off2, C)], idx_v0)
                pltpu.async_copy(combo_hbm.at[idx_v0], rows_v0, sem0)

            pltpu.make_async_copy(combo_hbm.at[idx_v1], rows_v1, sem1).wait()
            _dis_compute(rows_v1, acc_v)

        pltpu.sync_copy(acc_v, caus_hbm.at[wid])

        sbase = wid * SW
        pltpu.sync_copy(idx_hbm.at[pl.ds(DIS_ROWS + sbase, C)], idx_v0)
        pltpu.async_copy(combo_hbm.at[idx_v0], rows_v0, sem0)

        @pl.loop(0, NCHS, step=2)
        def _(g):
            off0 = sbase + g * C
            off1 = off0 + C
            pltpu.sync_copy(idx_hbm.at[pl.ds(DIS_ROWS + off1, C)], idx_v1)
            pltpu.async_copy(combo_hbm.at[idx_v1], rows_v1, sem1)
            pltpu.make_async_copy(combo_hbm.at[idx_v0], rows_v0, sem0).wait()
            pltpu.sync_copy(rows_v0, gsa_hbm.at[pl.ds(off0, C)])

            @pl.when(g + 2 < NCHS)
            def _():
                off2 = off0 + 2 * C
                pltpu.sync_copy(idx_hbm.at[pl.ds(DIS_ROWS + off2, C)], idx_v0)
                pltpu.async_copy(combo_hbm.at[idx_v0], rows_v0, sem0)

            pltpu.make_async_copy(combo_hbm.at[idx_v1], rows_v1, sem1).wait()
            pltpu.sync_copy(rows_v1, gsa_hbm.at[pl.ds(off1, C)])

    return k(combo, idx_all)


def kernel(states, next_states, dissimilar_pairs, same_actions_pairs, W):
    d0 = dissimilar_pairs[:, 0].astype(jnp.int32)
    d1 = dissimilar_pairs[:, 1].astype(jnp.int32)
    s0 = same_actions_pairs[:, 0].astype(jnp.int32)
    s1 = same_actions_pairs[:, 1].astype(jnp.int32)

    combo, temp_parts = _prep(states, next_states)

    parts_list = []
    caus_list = []
    for k in range(K):
        sl = slice(k * PK, (k + 1) * PK)
        dis_k = jnp.stack(
            [d0[sl].reshape(-1, HALF_C), d1[sl].reshape(-1, HALF_C)],
            axis=1).reshape(-1)
        idx_k = jnp.concatenate([dis_k, s0[sl], s1[sl]])
        g_sa, caus_parts = _sc_pass(combo, idx_k)
        parts_list.append(_loss(g_sa, W)[:, 0, :])
        caus_list.append(caus_parts)
    part_sum = jnp.sum(jnp.concatenate(parts_list, axis=0), axis=0)
    caus_sum = jnp.sum(jnp.concatenate(caus_list, axis=0))

    temp_coherence = jnp.sum(temp_parts[:, 0, 0]) / N
    causality = caus_sum / P
    proportionality = part_sum[1] / P
    repeatability = part_sum[2] / P
    l1 = part_sum[3] / (D * D * K)
    return (temp_coherence + causality + 5.0 * proportionality
            + 5.0 * repeatability + l1)


# speedup vs baseline: 1.0481x; 1.0481x over previous
import functools

import jax
import jax.numpy as jnp
from jax import lax
from jax.experimental import pallas as pl
from jax.experimental.pallas import tpu as pltpu
from jax.experimental.pallas import tpu_sc as plsc

N = 131072
D = 64
P = 262144

K = 4
PK = P // K
NUM_WORKERS = 32
C = 256
HALF_C = C // 2
DIS_ROWS = 2 * PK
DW = DIS_ROWS // NUM_WORKERS
NCHD = DW // C
SA_ROWS = 2 * PK
SW = SA_ROWS // NUM_WORKERS
NCHS = SW // C

PREP_R = 16384
LOSS_B = 8192


def _prep_body(st_ref, nt_ref, combo_ref, part_ref):
    st = st_ref[...]
    dt = nt_ref[...] - st
    combo_ref[:, :D] = st.T
    combo_ref[:, D:] = dt.T
    part_ref[...] = jnp.full((1, 1, 128), jnp.sum(dt * dt), dtype=jnp.float32)


def _loss_body(gsi_ref, gsj_ref, w_ref, out_ref):
    i = pl.program_id(0)
    lane = lax.broadcasted_iota(jnp.int32, (LOSS_B, 2 * D), 1)
    first = lane < D

    gsi = gsi_ref[...]
    gsj = gsj_ref[...]
    q = gsi - gsj
    q2 = q * q
    s2 = jnp.sum(jnp.where(first, q2, 0.0), axis=1)
    dd2 = jnp.sum(q2, axis=1) - s2
    rep = jnp.sum(jnp.exp(-s2) * dd2)

    gsi2 = gsi * gsi
    gsj2 = gsj * gsj
    na2 = jnp.sum(jnp.where(first, 0.0, gsi2), axis=1)
    nb2 = jnp.sum(jnp.where(first, 0.0, gsj2), axis=1)
    prop = jnp.sum(na2 + nb2 - 2.0 * jnp.sqrt(na2 * nb2))

    l1 = jnp.where(i == 0, jnp.sum(jnp.abs(w_ref[...])), 0.0)

    out_lane = lax.broadcasted_iota(jnp.int32, (1, 1, 128), 2)
    out_ref[...] = jnp.where(
        out_lane == 1, prop,
        jnp.where(out_lane == 2, rep,
                  jnp.where(out_lane == 3, l1, 0.0)))


def _prep(states, next_states):
    grid = N // PREP_R
    return pl.pallas_call(
        _prep_body,
        grid=(grid,),
        in_specs=[
            pl.BlockSpec((D, PREP_R), lambda i: (0, i)),
            pl.BlockSpec((D, PREP_R), lambda i: (0, i)),
        ],
        out_specs=[
            pl.BlockSpec((PREP_R, 2 * D), lambda i: (i, 0)),
            pl.BlockSpec((1, 1, 128), lambda i: (i, 0, 0)),
        ],
        out_shape=[
            jax.ShapeDtypeStruct((N, 2 * D), jnp.float32),
            jax.ShapeDtypeStruct((grid, 1, 128), jnp.float32),
        ],
        compiler_params=pltpu.CompilerParams(
            dimension_semantics=("parallel",)),
    )(states.T, next_states.T)


def _loss(g_sa, W):
    grid = PK // LOSS_B
    nb = grid
    return pl.pallas_call(
        _loss_body,
        grid=(grid,),
        in_specs=[
            pl.BlockSpec((LOSS_B, 2 * D), lambda i: (i, 0)),
            pl.BlockSpec((LOSS_B, 2 * D), lambda i: (nb + i, 0)),
            pl.BlockSpec((D, D), lambda i: (0, 0)),
        ],
        out_specs=pl.BlockSpec((1, 1, 128), lambda i: (i, 0, 0)),
        out_shape=jax.ShapeDtypeStruct((grid, 1, 128), jnp.float32),
        compiler_params=pltpu.CompilerParams(
            dimension_semantics=("parallel",)),
    )(g_sa, g_sa, W)


def _dis_compute(rows_ref, acc_ref):
    @pl.loop(0, HALF_C // 16)
    def _(grp):
        lane16 = lax.iota(jnp.int32, 16)
        gs2 = jnp.zeros((16,), jnp.float32)
        for b in range(16):
            p = grp * 16 + b
            acc = None
            for c in range(D // 16):
                vi = rows_ref[p, pl.ds(c * 16, 16)]
                vj = rows_ref[p + HALF_C, pl.ds(c * 16, 16)]
                dlt = vi - vj
                acc = dlt * dlt if acc is None else acc + dlt * dlt
            s = jnp.sum(acc)
            gs2 = jnp.where(lane16 == b, s, gs2)
        acc_ref[...] = acc_ref[...] + jnp.exp(-gs2)


def _sc_pass(combo, idx_all):
    mesh = plsc.VectorSubcoreMesh(core_axis_name="c", subcore_axis_name="s")

    @functools.partial(
        pl.kernel,
        out_type=[
            jax.ShapeDtypeStruct((SA_ROWS, 2 * D), jnp.float32),
            jax.ShapeDtypeStruct((NUM_WORKERS, 16), jnp.float32),
        ],
        mesh=mesh,
        scratch_types=[
            pltpu.VMEM((C,), jnp.int32),
            pltpu.VMEM((C,), jnp.int32),
            pltpu.VMEM((C, 2 * D), jnp.float32),
            pltpu.VMEM((C, 2 * D), jnp.float32),
            pltpu.VMEM((16,), jnp.float32),
            pltpu.SemaphoreType.DMA,
            pltpu.SemaphoreType.DMA,
        ],
        compiler_params=pltpu.CompilerParams(needs_layout_passes=False),
    )
    def k(combo_hbm, idx_hbm, gsa_hbm, caus_hbm,
          idx_v0, idx_v1, rows_v0, rows_v1, acc_v, sem0, sem1):
        wid = lax.axis_index("s") * 2 + lax.axis_index("c")

        dbase = wid * DW
        acc_v[...] = jnp.zeros((16,), jnp.float32)
        pltpu.sync_copy(idx_hbm.at[pl.ds(dbase, C)], idx_v0)
        pltpu.async_copy(combo_hbm.at[idx_v0], rows_v0, sem0)

        @pl.loop(0, NCHD, step=2)
        def _(g):
            off1 = dbase + (g + 1) * C
            pltpu.sync_copy(idx_hbm.at[pl.ds(off1, C)], idx_v1)
            pltpu.async_copy(combo_hbm.at[idx_v1], rows_v1, sem1)
            pltpu.make_async_copy(combo_hbm.at[idx_v0], rows_v0, sem0).wait()
            _dis_compute(rows_v0, acc_v)

            @pl.when(g + 2 < NCHD)
            def _():
                off2 = dbase + (g + 2) * C
                pltpu.sync_copy(idx_hbm.at[pl.ds(off2, C)], idx_v0)
                pltpu.async_copy(combo_hbm.at[idx_v0], rows_v0, sem0)

            pltpu.make_async_copy(combo_hbm.at[idx_v1], rows_v1, sem1).wait()
            _dis_compute(rows_v1, acc_v)

        pltpu.sync_copy(acc_v, caus_hbm.at[wid])

        sbase = wid * SW
        pltpu.sync_copy(idx_hbm.at[pl.ds(DIS_ROWS + sbase, C)], idx_v0)
        pltpu.async_copy(combo_hbm.at[idx_v0], rows_v0, sem0)

        @pl.loop(0, NCHS, step=2)
        def _(g):
            off0 = sbase + g * C
            off1 = off0 + C
            pltpu.sync_copy(idx_hbm.at[pl.ds(DIS_ROWS + off1, C)], idx_v1)
            pltpu.async_copy(combo_hbm.at[idx_v1], rows_v1, sem1)
            pltpu.make_async_copy(combo_hbm.at[idx_v0], rows_v0, sem0).wait()
            pltpu.sync_copy(rows_v0, gsa_hbm.at[pl.ds(off0, C)])

            @pl.when(g + 2 < NCHS)
            def _():
                off2 = off0 + 2 * C
                pltpu.sync_copy(idx_hbm.at[pl.ds(DIS_ROWS + off2, C)], idx_v0)
                pltpu.async_copy(combo_hbm.at[idx_v0], rows_v0, sem0)

            pltpu.make_async_copy(combo_hbm.at[idx_v1], rows_v1, sem1).wait()
            pltpu.sync_copy(rows_v1, gsa_hbm.at[pl.ds(off1, C)])

    return k(combo, idx_all)


def kernel(states, next_states, dissimilar_pairs, same_actions_pairs, W):
    d0 = dissimilar_pairs[:, 0].astype(jnp.int32)
    d1 = dissimilar_pairs[:, 1].astype(jnp.int32)
    s0 = same_actions_pairs[:, 0].astype(jnp.int32)
    s1 = same_actions_pairs[:, 1].astype(jnp.int32)

    combo, temp_parts = _prep(states, next_states)

    parts_list = []
    caus_list = []
    for k in range(K):
        sl = slice(k * PK, (k + 1) * PK)
        dis_k = jnp.stack(
            [d0[sl].reshape(-1, HALF_C), d1[sl].reshape(-1, HALF_C)],
            axis=1).reshape(-1)
        idx_k = jnp.concatenate([dis_k, s0[sl], s1[sl]])
        g_sa, caus_parts = _sc_pass(combo, idx_k)
        parts_list.append(_loss(g_sa, W)[:, 0, :])
        caus_list.append(caus_parts)
    part_sum = jnp.sum(jnp.concatenate(parts_list, axis=0), axis=0)
    caus_sum = jnp.sum(jnp.concatenate(caus_list, axis=0))

    temp_coherence = jnp.sum(temp_parts[:, 0, 0]) / N
    causality = caus_sum / P
    proportionality = part_sum[1] / P
    repeatability = part_sum[2] / P
    l1 = part_sum[3] / (D * D * K)
    return (temp_coherence + causality + 5.0 * proportionality
            + 5.0 * repeatability + l1)
